# Initial kernel scaffold; baseline (speedup 1.0000x reference)
import jax
import jax.numpy as jnp
from jax.experimental import pallas as pl


def kernel(x, Wr, br, W1, b1, W2, b2):
    raise NotImplementedError("write your pallas kernel here")


# speedup vs baseline: 5.2706x; 5.2706x over previous
import functools

import jax
import jax.numpy as jnp
from jax import lax
from jax.experimental import pallas as pl
from jax.experimental.pallas import tpu as pltpu
from jax.experimental.pallas import tpu_sc as plsc

N = 2048
D = 768
F = 1536
E = 64
TM = 128
G_MAX = N // TM + E - 1
P_MAX = G_MAX * TM
NW = 32
CHUNK = N // NW


def _router_body(x_ref, wr_ref, br_ref, idx_ref, w_ref, rank_ref, cnt_ref,
                 oh_ref, cs_ref):
    x = x_ref[...]
    logits = lax.dot_general(x, wr_ref[...], (((1,), (1,)), ((), ())),
                             preferred_element_type=jnp.float32) + br_ref[...]
    lmax = jnp.max(logits, axis=1, keepdims=True)
    ee = jnp.exp(logits - lmax)
    probs = ee / jnp.sum(ee, axis=1, keepdims=True)
    pmax = jnp.max(probs, axis=1, keepdims=True)
    eids = lax.broadcasted_iota(jnp.int32, (N, E), 1)
    idx = jnp.min(jnp.where(probs >= pmax, eids, E), axis=1, keepdims=True)
    oh_ref[...] = (eids == idx).astype(jnp.float32)

    tri = (lax.broadcasted_iota(jnp.int32, (TM, TM), 1)
           <= lax.broadcasted_iota(jnp.int32, (TM, TM), 0)).astype(jnp.float32)

    def blk(b, tot):
        oh = oh_ref[pl.ds(b * TM, TM), :]
        c = lax.dot_general(tri, oh, (((1,), (0,)), ((), ())),
                            preferred_element_type=jnp.float32) + tot
        cs_ref[pl.ds(b * TM, TM), :] = c
        return c[TM - 1:TM, :]

    tot = lax.fori_loop(0, N // TM, blk, jnp.zeros((1, E), jnp.float32))
    rank = jnp.sum(oh_ref[...] * cs_ref[...], axis=1, keepdims=True) - 1.0
    idx_ref[...] = idx
    w_ref[...] = pmax
    rank_ref[...] = rank.astype(jnp.int32)
    cnt_ref[...] = tot.astype(jnp.int32)


_router = pl.pallas_call(
    _router_body,
    out_shape=(
        jax.ShapeDtypeStruct((N, 1), jnp.int32),
        jax.ShapeDtypeStruct((N, 1), jnp.float32),
        jax.ShapeDtypeStruct((N, 1), jnp.int32),
        jax.ShapeDtypeStruct((1, E), jnp.int32),
    ),
    scratch_shapes=[pltpu.VMEM((N, E), jnp.float32),
                    pltpu.VMEM((N, E), jnp.float32)],
)


def _dispatch_body(x_hbm, dest_hbm, xpad_hbm, idx_v, rows_v, sem):
    wid = lax.axis_index("s") * 2 + lax.axis_index("c")
    base = wid * CHUNK
    pltpu.sync_copy(dest_hbm.at[pl.ds(base, CHUNK)], idx_v)
    pltpu.sync_copy(x_hbm.at[pl.ds(base, CHUNK)], rows_v)
    pltpu.async_copy(rows_v, xpad_hbm.at[idx_v], sem).wait()


_dispatch = functools.partial(
    pl.kernel,
    mesh=plsc.VectorSubcoreMesh(core_axis_name="c", subcore_axis_name="s"),
    out_type=jax.ShapeDtypeStruct((P_MAX, D), jnp.float32),
    scratch_types=[pltpu.VMEM((CHUNK,), jnp.int32),
                   pltpu.VMEM((CHUNK, D), jnp.float32),
                   pltpu.SemaphoreType.DMA],
)(_dispatch_body)


def _gmm_body(se_ref, st_ref, vd_ref, x_ref, w1_ref, b1_ref, w2_ref, b2_ref,
              o_ref):
    @pl.when(vd_ref[pl.program_id(0)] == 1)
    def _():
        xg = x_ref[...]
        h = jnp.maximum(
            lax.dot_general(xg, w1_ref[0], (((1,), (1,)), ((), ())),
                            preferred_element_type=jnp.float32)
            + b1_ref[0], 0.0)
        o_ref[...] = lax.dot_general(h, w2_ref[0], (((1,), (1,)), ((), ())),
                                     preferred_element_type=jnp.float32) + b2_ref[0]


_gmm = pl.pallas_call(
    _gmm_body,
    grid_spec=pltpu.PrefetchScalarGridSpec(
        num_scalar_prefetch=3,
        grid=(G_MAX,),
        in_specs=[
            pl.BlockSpec((TM, D), lambda g, se, st, vd: (st[g], 0)),
            pl.BlockSpec((1, F, D), lambda g, se, st, vd: (se[g], 0, 0)),
            pl.BlockSpec((1, 1, F), lambda g, se, st, vd: (se[g], 0, 0)),
            pl.BlockSpec((1, D, F), lambda g, se, st, vd: (se[g], 0, 0)),
            pl.BlockSpec((1, 1, D), lambda g, se, st, vd: (se[g], 0, 0)),
        ],
        out_specs=pl.BlockSpec((TM, D), lambda g, se, st, vd: (st[g], 0)),
    ),
    out_shape=jax.ShapeDtypeStruct((P_MAX, D), jnp.float32),
    compiler_params=pltpu.CompilerParams(
        dimension_semantics=("arbitrary",)),
)


def _combine_body(opad_hbm, dest_hbm, win_hbm, y_hbm, idx_v, w_v, rows_v, sem):
    wid = lax.axis_index("s") * 2 + lax.axis_index("c")
    base = wid * CHUNK
    pltpu.sync_copy(dest_hbm.at[pl.ds(base, CHUNK)], idx_v)
    pltpu.sync_copy(win_hbm.at[pl.ds(base, CHUNK)], w_v)
    pltpu.async_copy(opad_hbm.at[idx_v], rows_v, sem).wait()

    def row(r, carry):
        w16 = w_v[r]
        for c in range(D // 16):
            rows_v[r, pl.ds(c * 16, 16)] = rows_v[r, pl.ds(c * 16, 16)] * w16
        return carry

    lax.fori_loop(0, CHUNK, row, 0)
    pltpu.sync_copy(rows_v, y_hbm.at[pl.ds(base, CHUNK)])


_combine = functools.partial(
    pl.kernel,
    mesh=plsc.VectorSubcoreMesh(core_axis_name="c", subcore_axis_name="s"),
    out_type=jax.ShapeDtypeStruct((N, D), jnp.float32),
    scratch_types=[pltpu.VMEM((CHUNK,), jnp.int32),
                   pltpu.VMEM((CHUNK, 16), jnp.float32),
                   pltpu.VMEM((CHUNK, D), jnp.float32),
                   pltpu.SemaphoreType.DMA],
)(_combine_body)


def kernel(x, Wr, br, W1, b1, W2, b2):
    idx2, w2g, rank2, cnt2 = _router(x, Wr, br.reshape(1, E))
    idx = idx2[:, 0]
    rank = rank2[:, 0]
    counts = cnt2[0]

    tiles_e = (counts + TM - 1) // TM
    bounds = jnp.cumsum(tiles_e)
    n_tiles = bounds[-1]
    pad_off = (bounds - tiles_e) * TM
    dest = pad_off[idx] + rank
    g = jnp.arange(G_MAX, dtype=jnp.int32)
    sched_tile = jnp.minimum(g, n_tiles - 1)
    sched_expert = jnp.sum((bounds[None, :] <= sched_tile[:, None])
                           .astype(jnp.int32), axis=1)
    valid = (g < n_tiles).astype(jnp.int32)

    xpad = _dispatch(x, dest)
    opad = _gmm(sched_expert, sched_tile, valid, xpad,
                W1, b1.reshape(E, 1, F), W2, b2.reshape(E, 1, D))
    win = jnp.broadcast_to(w2g, (N, 16))
    return _combine(opad, dest, win)


# speedup vs baseline: 6.7337x; 1.2776x over previous
import functools

import jax
import jax.numpy as jnp
from jax import lax
from jax.experimental import pallas as pl
from jax.experimental.pallas import tpu as pltpu
from jax.experimental.pallas import tpu_sc as plsc

N = 2048
D = 768
F = 1536
E = 64
TM = 128
G_MAX = N // TM + E - 1
G_PAD = 128
P_MAX = G_MAX * TM
NW = 32
CHUNK = N // NW


def _router_body(x_ref, wr_ref, br_ref,
                 dest_ref, win_ref, se_ref, st_ref, vd_ref,
                 oh_ref, cs_ref):
    x = x_ref[...]
    logits = lax.dot_general(x, wr_ref[...], (((1,), (1,)), ((), ())),
                             preferred_element_type=jnp.float32) + br_ref[...]
    lmax = jnp.max(logits, axis=1, keepdims=True)
    ee = jnp.exp(logits - lmax)
    probs = ee / jnp.sum(ee, axis=1, keepdims=True)
    pmax = jnp.max(probs, axis=1, keepdims=True)
    eids = lax.broadcasted_iota(jnp.int32, (N, E), 1)
    idx = jnp.min(jnp.where(probs >= pmax, eids, E), axis=1, keepdims=True)
    onehot = (eids == idx).astype(jnp.float32)
    oh_ref[...] = onehot

    tri = (lax.broadcasted_iota(jnp.int32, (TM, TM), 1)
           <= lax.broadcasted_iota(jnp.int32, (TM, TM), 0)).astype(jnp.float32)

    def blk(b, tot):
        oh = oh_ref[pl.ds(b * TM, TM), :]
        c = lax.dot_general(tri, oh, (((1,), (0,)), ((), ())),
                            preferred_element_type=jnp.float32) + tot
        cs_ref[pl.ds(b * TM, TM), :] = c
        return c[TM - 1:TM, :]

    tot = lax.fori_loop(0, N // TM, blk, jnp.zeros((1, E), jnp.float32))
    rank = jnp.sum(onehot * cs_ref[...], axis=1, keepdims=True) - 1.0

    counts = tot.astype(jnp.int32)
    tiles_e = lax.shift_right_logical(counts + (TM - 1), 7)
    triE = (lax.broadcasted_iota(jnp.int32, (E, E), 0)
            <= lax.broadcasted_iota(jnp.int32, (E, E), 1)).astype(jnp.float32)
    bounds = lax.dot_general(tiles_e.astype(jnp.float32), triE,
                             (((1,), (0,)), ((), ())),
                             preferred_element_type=jnp.float32)
    n_tiles = bounds[0, E - 1].astype(jnp.int32)
    pad_off = (bounds - tiles_e.astype(jnp.float32)) * float(TM)
    dest = jnp.sum(onehot * pad_off, axis=1, keepdims=True) + rank
    dest_ref[...] = dest.astype(jnp.int32)
    win_ref[...] = jnp.broadcast_to(pmax, (N, 128))

    g_col = lax.broadcasted_iota(jnp.int32, (G_PAD, 1), 0)
    st = jnp.minimum(g_col, n_tiles - 1)
    st_ref[...] = st
    vd_ref[...] = (g_col < n_tiles).astype(jnp.int32)
    se_ref[...] = jnp.sum((bounds <= st.astype(jnp.float32))
                          .astype(jnp.int32), axis=1, keepdims=True)


_router = pl.pallas_call(
    _router_body,
    out_shape=(
        jax.ShapeDtypeStruct((N, 1), jnp.int32),
        jax.ShapeDtypeStruct((N, 128), jnp.float32),
        jax.ShapeDtypeStruct((G_PAD, 1), jnp.int32),
        jax.ShapeDtypeStruct((G_PAD, 1), jnp.int32),
        jax.ShapeDtypeStruct((G_PAD, 1), jnp.int32),
    ),
    scratch_shapes=[pltpu.VMEM((N, E), jnp.float32),
                    pltpu.VMEM((N, E), jnp.float32)],
)


def _dispatch_body(x_hbm, dest_hbm, win_hbm, xpad_hbm, wpad_hbm,
                   idx_v, rows_v, wrow_v, sem):
    wid = lax.axis_index("s") * 2 + lax.axis_index("c")
    base = wid * CHUNK
    pltpu.sync_copy(dest_hbm.at[pl.ds(base, CHUNK)], idx_v)
    pltpu.sync_copy(x_hbm.at[pl.ds(base, CHUNK)], rows_v)
    pltpu.sync_copy(win_hbm.at[pl.ds(base, CHUNK)], wrow_v)
    c1 = pltpu.async_copy(rows_v, xpad_hbm.at[idx_v], sem)
    c2 = pltpu.async_copy(wrow_v, wpad_hbm.at[idx_v], sem)
    c1.wait()
    c2.wait()


_dispatch = functools.partial(
    pl.kernel,
    mesh=plsc.VectorSubcoreMesh(core_axis_name="c", subcore_axis_name="s"),
    out_type=(jax.ShapeDtypeStruct((P_MAX, D), jnp.float32),
              jax.ShapeDtypeStruct((P_MAX, 128), jnp.float32)),
    scratch_types=[pltpu.VMEM((CHUNK,), jnp.int32),
                   pltpu.VMEM((CHUNK, D), jnp.float32),
                   pltpu.VMEM((CHUNK, 128), jnp.float32),
                   pltpu.SemaphoreType.DMA],
)(_dispatch_body)


def _gmm_body(se_ref, st_ref, vd_ref, x_ref, wp_ref, w1_ref, b1_ref, w2_ref,
              b2_ref, o_ref):
    @pl.when(vd_ref[pl.program_id(0)] == 1)
    def _():
        xg = x_ref[...]
        h = jnp.maximum(
            lax.dot_general(xg, w1_ref[0], (((1,), (1,)), ((), ())),
                            preferred_element_type=jnp.float32)
            + b1_ref[0], 0.0)
        eo = lax.dot_general(h, w2_ref[0], (((1,), (1,)), ((), ())),
                             preferred_element_type=jnp.float32) + b2_ref[0]
        o_ref[...] = eo * wp_ref[:, 0:1]


_gmm = pl.pallas_call(
    _gmm_body,
    grid_spec=pltpu.PrefetchScalarGridSpec(
        num_scalar_prefetch=3,
        grid=(G_MAX,),
        in_specs=[
            pl.BlockSpec((TM, D), lambda g, se, st, vd: (st[g], 0)),
            pl.BlockSpec((TM, 128), lambda g, se, st, vd: (st[g], 0)),
            pl.BlockSpec((1, F, D), lambda g, se, st, vd: (se[g], 0, 0)),
            pl.BlockSpec((1, 1, F), lambda g, se, st, vd: (se[g], 0, 0)),
            pl.BlockSpec((1, D, F), lambda g, se, st, vd: (se[g], 0, 0)),
            pl.BlockSpec((1, 1, D), lambda g, se, st, vd: (se[g], 0, 0)),
        ],
        out_specs=pl.BlockSpec((TM, D), lambda g, se, st, vd: (st[g], 0)),
    ),
    out_shape=jax.ShapeDtypeStruct((P_MAX, D), jnp.float32),
    compiler_params=pltpu.CompilerParams(
        dimension_semantics=("arbitrary",)),
)


def _combine_body(opad_hbm, dest_hbm, y_hbm, idx_v, rows_v, sem):
    wid = lax.axis_index("s") * 2 + lax.axis_index("c")
    base = wid * CHUNK
    pltpu.sync_copy(dest_hbm.at[pl.ds(base, CHUNK)], idx_v)
    pltpu.async_copy(opad_hbm.at[idx_v], rows_v, sem).wait()
    pltpu.sync_copy(rows_v, y_hbm.at[pl.ds(base, CHUNK)])


_combine = functools.partial(
    pl.kernel,
    mesh=plsc.VectorSubcoreMesh(core_axis_name="c", subcore_axis_name="s"),
    out_type=jax.ShapeDtypeStruct((N, D), jnp.float32),
    scratch_types=[pltpu.VMEM((CHUNK,), jnp.int32),
                   pltpu.VMEM((CHUNK, D), jnp.float32),
                   pltpu.SemaphoreType.DMA],
)(_combine_body)


def kernel(x, Wr, br, W1, b1, W2, b2):
    dest2, win, se2, st2, vd2 = _router(x, Wr, br.reshape(1, E))
    dest = dest2.reshape(N)
    xpad, wpad = _dispatch(x, dest, win)
    opad = _gmm(se2.reshape(G_PAD), st2.reshape(G_PAD), vd2.reshape(G_PAD),
                xpad, wpad, W1, b1.reshape(E, 1, F), W2, b2.reshape(E, 1, D))
    return _combine(opad, dest)
